# baseline (speedup 1.0000x reference)
import functools

import jax
import jax.numpy as jnp
from jax.experimental import pallas as pl
from jax.experimental.pallas import tpu as pltpu


def _gating_kernel(x_ref, w1_ref, b1_ref, w2_ref, b2_ref,
                   wts_ref, idx_ref, aux_ref,
                   h_acc, logit_acc, usage_acc,
                   *, nm, nk, nn, bm, bn, n_experts, top_k, tokens):
    m = pl.program_id(0)
    k = pl.program_id(1)
    n = pl.program_id(2)

    part_h = jnp.dot(x_ref[...], w1_ref[...], preferred_element_type=jnp.float32)
    nds = pl.ds(n * bn, bn)

    @pl.when(k == 0)
    def _():
        h_acc[:, nds] = part_h

    @pl.when(k != 0)
    def _():
        h_acc[:, nds] += part_h

    @pl.when(k == nk - 1)
    def _second_stage():
        h = h_acc[:, nds] + b1_ref[...]
        g = 0.5 * h * (1.0 + jax.lax.erf(h * 0.7071067811865476))
        part = jnp.dot(g, w2_ref[...], preferred_element_type=jnp.float32)

        @pl.when(n == 0)
        def _():
            logit_acc[...] = part

        @pl.when(n != 0)
        def _():
            logit_acc[...] += part

        @pl.when(n == nn - 1)
        def _epilogue():
            lt = (logit_acc[...] + b2_ref[...]).T
            iota_e = jax.lax.broadcasted_iota(jnp.int32, (n_experts, bm), 0)
            cur = lt
            vals = []
            idxs = []
            for _ in range(top_k):
                v = jnp.max(cur, axis=0, keepdims=True)
                i = jnp.min(jnp.where(cur == v, iota_e, n_experts),
                            axis=0, keepdims=True)
                vals.append(v)
                idxs.append(i)
                cur = jnp.where(iota_e == i, -1e30, cur)
            topv = jnp.concatenate(vals, axis=0)
            topi = jnp.concatenate(idxs, axis=0)
            exps = jnp.exp(topv - vals[0])
            wts_t = exps / jnp.sum(exps, axis=0, keepdims=True)
            wts_ref[...] = wts_t.T
            idx_ref[...] = topi.T
            p = jnp.exp(lt - vals[0])
            p = p / jnp.sum(p, axis=0, keepdims=True)
            colsum = jnp.sum(p, axis=1, keepdims=True).T

            @pl.when(m == 0)
            def _():
                usage_acc[...] = colsum

            @pl.when(m != 0)
            def _():
                usage_acc[...] += colsum

            @pl.when(m == nm - 1)
            def _final():
                usage = usage_acc[...] / tokens
                diff = usage - (1.0 / n_experts)
                aux_ref[...] = jnp.sum(diff * diff, keepdims=True).reshape(1, 1)


def kernel(x, W1, b1, W2, b2, training):
    tokens, d_model = x.shape
    hidden = W1.shape[1]
    n_experts = W2.shape[1]
    top_k = 8

    bm = min(2048, tokens)
    bk = min(512, d_model)
    bn = min(512, hidden)
    nm = tokens // bm
    nk = d_model // bk
    nn = hidden // bn

    b1r = b1.reshape(1, hidden)
    b2r = b2.reshape(1, n_experts)

    body = functools.partial(_gating_kernel, nm=nm, nk=nk, nn=nn, bm=bm,
                             bn=bn, n_experts=n_experts, top_k=top_k,
                             tokens=tokens)

    wts, idx, aux = pl.pallas_call(
        body,
        grid=(nm, nk, nn),
        in_specs=[
            pl.BlockSpec((bm, bk), lambda m, k, n: (m, k)),
            pl.BlockSpec((bk, bn), lambda m, k, n: (k, n)),
            pl.BlockSpec((1, bn), lambda m, k, n: (0, n)),
            pl.BlockSpec((bn, n_experts), lambda m, k, n: (n, 0)),
            pl.BlockSpec((1, n_experts), lambda m, k, n: (0, 0)),
        ],
        out_specs=[
            pl.BlockSpec((bm, top_k), lambda m, k, n: (m, 0)),
            pl.BlockSpec((bm, top_k), lambda m, k, n: (m, 0)),
            pl.BlockSpec((1, 1), lambda m, k, n: (0, 0)),
        ],
        out_shape=[
            jax.ShapeDtypeStruct((tokens, top_k), jnp.float32),
            jax.ShapeDtypeStruct((tokens, top_k), jnp.int32),
            jax.ShapeDtypeStruct((1, 1), jnp.float32),
        ],
        scratch_shapes=[
            pltpu.VMEM((bm, hidden), jnp.float32),
            pltpu.VMEM((bm, n_experts), jnp.float32),
            pltpu.VMEM((1, n_experts), jnp.float32),
        ],
    )(x, W1, b1r, W2, b2r)

    return wts, idx, aux[0, 0]


# speedup vs baseline: 1.4200x; 1.4200x over previous
import functools

import jax
import jax.numpy as jnp
from jax.experimental import pallas as pl
from jax.experimental.pallas import tpu as pltpu


def _gating_kernel(x_hbm, w1_ref, b1_ref, w2_ref, b2_ref,
                   wts_ref, idx_ref, aux_ref,
                   x_tile, logit_acc, usage_acc, dma_sem,
                   *, nm, nn, bm, n_experts, top_k, tokens):
    m = pl.program_id(0)
    n = pl.program_id(1)

    @pl.when(n == 0)
    def _load_x():
        cp = pltpu.make_async_copy(
            x_hbm.at[pl.ds(m * bm, bm), :], x_tile, dma_sem)
        cp.start()
        cp.wait()

    h = jnp.dot(x_tile[...], w1_ref[...], preferred_element_type=jnp.float32)
    h = h + b1_ref[...]
    g = 0.5 * h * (1.0 + jax.lax.erf(h * 0.7071067811865476))
    part = jnp.dot(g, w2_ref[...], preferred_element_type=jnp.float32)

    @pl.when(n == 0)
    def _():
        logit_acc[...] = part

    @pl.when(n != 0)
    def _():
        logit_acc[...] += part

    @pl.when(n == nn - 1)
    def _epilogue():
        lt = (logit_acc[...] + b2_ref[...]).T
        iota_e = jax.lax.broadcasted_iota(jnp.int32, (n_experts, bm), 0)
        cur = lt
        vals = []
        idxs = []
        for _ in range(top_k):
            v = jnp.max(cur, axis=0, keepdims=True)
            i = jnp.min(jnp.where(cur == v, iota_e, n_experts),
                        axis=0, keepdims=True)
            vals.append(v)
            idxs.append(i)
            cur = jnp.where(iota_e == i, -1e30, cur)
        topv = jnp.concatenate(vals, axis=0)
        topi = jnp.concatenate(idxs, axis=0)
        exps = jnp.exp(topv - vals[0])
        wts_t = exps / jnp.sum(exps, axis=0, keepdims=True)
        wts_ref[...] = wts_t
        idx_ref[...] = topi
        p = jnp.exp(lt - vals[0])
        p = p / jnp.sum(p, axis=0, keepdims=True)
        colsum = jnp.sum(p, axis=1, keepdims=True).T

        @pl.when(m == 0)
        def _():
            usage_acc[...] = colsum

        @pl.when(m != 0)
        def _():
            usage_acc[...] += colsum

        @pl.when(m == nm - 1)
        def _final():
            usage = usage_acc[...] / tokens
            diff = usage - (1.0 / n_experts)
            aux_ref[...] = jnp.sum(diff * diff, keepdims=True).reshape(1, 1)


def kernel(x, W1, b1, W2, b2, training):
    tokens, d_model = x.shape
    hidden = W1.shape[1]
    n_experts = W2.shape[1]
    top_k = 8

    bm = min(2048, tokens)
    bn = min(512, hidden)
    nm = tokens // bm
    nn = hidden // bn

    b1r = b1.reshape(1, hidden)
    b2r = b2.reshape(1, n_experts)

    body = functools.partial(_gating_kernel, nm=nm, nn=nn, bm=bm,
                             n_experts=n_experts, top_k=top_k, tokens=tokens)

    wts, idx, aux = pl.pallas_call(
        body,
        grid=(nm, nn),
        in_specs=[
            pl.BlockSpec(memory_space=pl.ANY),
            pl.BlockSpec((d_model, bn), lambda m, n: (0, n)),
            pl.BlockSpec((1, bn), lambda m, n: (0, n)),
            pl.BlockSpec((bn, n_experts), lambda m, n: (n, 0)),
            pl.BlockSpec((1, n_experts), lambda m, n: (0, 0)),
        ],
        out_specs=[
            pl.BlockSpec((top_k, bm), lambda m, n: (0, m)),
            pl.BlockSpec((top_k, bm), lambda m, n: (0, m)),
            pl.BlockSpec((1, 1), lambda m, n: (0, 0)),
        ],
        out_shape=[
            jax.ShapeDtypeStruct((top_k, tokens), jnp.float32),
            jax.ShapeDtypeStruct((top_k, tokens), jnp.int32),
            jax.ShapeDtypeStruct((1, 1), jnp.float32),
        ],
        scratch_shapes=[
            pltpu.VMEM((bm, d_model), jnp.float32),
            pltpu.VMEM((bm, n_experts), jnp.float32),
            pltpu.VMEM((1, n_experts), jnp.float32),
            pltpu.SemaphoreType.DMA,
        ],
    )(x, W1, b1r, W2, b2r)

    return wts.T, idx.T, aux[0, 0]
